# baseline (speedup 1.0000x reference)
import functools

import jax
import jax.numpy as jnp
from jax import lax
from jax.experimental import pallas as pl
from jax.experimental.pallas import tpu as pltpu
from jax.experimental.pallas import tpu_sc as plsc

_NS = 16
_RPW = 8
_K = 4096
_KV = _K // 16

_mesh = plsc.VectorSubcoreMesh(
    core_axis_name="c", subcore_axis_name="s", num_cores=1)


def _rsqrt_vec(v):
    i = plsc.bitcast(v, jnp.int32)
    i = jnp.full((16,), 0x5F3759DF, jnp.int32) - lax.shift_right_logical(
        i, jnp.full((16,), 1, jnp.int32))
    y = plsc.bitcast(i, jnp.float32)
    half = v * 0.5
    for _ in range(4):
        y = y * (1.5 - half * y * y)
    return y


def _sc_body(z_hbm, w1_hbm, b1_hbm, a_hbm, lnw_hbm, lnb_hbm,
             w2_hbm, b2_hbm, out_hbm,
             rows_v, x_v, a_v, b1_v, lnw_v, lnb_v, w2_v, b2_v,
             hseg_v, lseg_v, ep_v, lg_v, out_v, shared_h, shared_l,
             sem_x, sem_rows):
    s = lax.axis_index("s")
    lane = lax.iota(jnp.int32, 16)

    cp_x = pltpu.make_async_copy(z_hbm, x_v, sem_x)
    cp_x.start()
    cp_rows = pltpu.make_async_copy(
        w1_hbm.at[s, :, pl.ds(0, _K)], rows_v, sem_rows)
    cp_rows.start()

    pltpu.sync_copy(a_hbm, a_v.at[pl.ds(0, 1)])
    pltpu.sync_copy(b1_hbm, b1_v)
    pltpu.sync_copy(lnw_hbm, lnw_v)
    pltpu.sync_copy(lnb_hbm, lnb_v)
    pltpu.sync_copy(b2_hbm, b2_v)

    cp_x.wait()
    cp_rows.wait()

    a_s = a_v[pl.ds(0, 16)][0]

    accs = [jnp.zeros((16,), jnp.float32) for _ in range(_RPW)]
    for k in range(_KV):
        xk = x_v[pl.ds(k * 16, 16)]
        for j in range(_RPW):
            accs[j] = accs[j] + xk * rows_v[j, pl.ds(k * 16, 16)]
    hvec = jnp.zeros((16,), jnp.float32)
    for j in range(_RPW):
        hvec = jnp.where(lane == j, jnp.sum(accs[j]), hvec)

    b1g = plsc.load_gather(b1_v, [s * _RPW + (lane & 7)])
    hvec = hvec + b1g
    av = jnp.full((16,), a_s, jnp.float32)
    hvec = jnp.maximum(hvec, 0.0) + av * jnp.minimum(hvec, 0.0)

    hseg_v[...] = hvec
    pltpu.sync_copy(hseg_v, shared_h.at[pl.ds(s * 16, 16)])
    plsc.subcore_barrier()

    pltpu.sync_copy(shared_h, ep_v)
    hs = []
    for m in range(8):
        gidx = m * 16 + lane
        hs.append(plsc.load_gather(ep_v, [((gidx >> 3) << 4) + (gidx & 7)]))
    hsum = hs[0]
    sumsq = hs[0] * hs[0]
    for m in range(1, 8):
        hsum = hsum + hs[m]
        sumsq = sumsq + hs[m] * hs[m]
    mu = jnp.sum(hsum) * (1.0 / 128.0)
    var = jnp.sum(sumsq) * (1.0 / 128.0) - mu * mu
    rstd = _rsqrt_vec(jnp.full((16,), var + 1e-5, jnp.float32))
    mu_vec = jnp.full((16,), mu, jnp.float32)
    hn = []
    for m in range(8):
        hn.append((hs[m] - mu_vec) * rstd
                  * lnw_v[pl.ds(m * 16, 16)] + lnb_v[pl.ds(m * 16, 16)])

    @pl.when(s < 8)
    def _logits():
        pltpu.sync_copy(w2_hbm.at[pl.ds(s * 8, 8), :], w2_v)
        lvec = jnp.zeros((16,), jnp.float32)
        for i in range(8):
            acc = hn[0] * w2_v[i, pl.ds(0, 16)]
            for m in range(1, 8):
                acc = acc + hn[m] * w2_v[i, pl.ds(m * 16, 16)]
            lvec = jnp.where(lane == i, jnp.sum(acc), lvec)
        b2g = plsc.load_gather(b2_v, [s * 8 + (lane & 7)])
        lseg_v[...] = lvec + b2g
        pltpu.sync_copy(lseg_v, shared_l.at[pl.ds(s * 16, 16)])

    plsc.subcore_barrier()

    @pl.when(s == 0)
    def _epilogue():
        pltpu.sync_copy(shared_l, lg_v)
        ls = []
        for t in range(4):
            gidx = t * 16 + lane
            ls.append(plsc.load_gather(lg_v, [((gidx >> 3) << 4) + (gidx & 7)]))
        mx = jnp.maximum(jnp.maximum(ls[0], ls[1]), jnp.maximum(ls[2], ls[3]))
        mxs = jnp.full((16,), jnp.max(mx), jnp.float32)
        es = [jnp.exp(l - mxs) for l in ls]
        tot = jnp.sum(es[0] + es[1] + es[2] + es[3])
        inv = jnp.full((16,), 1.0, jnp.float32) / jnp.full((16,), tot, jnp.float32)
        for t in range(4):
            out_v[pl.ds(t * 16, 16)] = es[t] * inv
        pltpu.sync_copy(out_v, out_hbm)


@jax.jit
def _gate(z1d, W1v, b1_eff, a, lnw, lnb, W2, b2):
    f32 = jnp.float32
    run = functools.partial(
        pl.kernel,
        out_type=jax.ShapeDtypeStruct((64,), f32),
        mesh=_mesh,
        compiler_params=pltpu.CompilerParams(
            needs_layout_passes=False, use_tc_tiling_on_sc=True),
        scratch_types=[
            pltpu.VMEM((_RPW, _K), f32),
            pltpu.VMEM((_K,), f32),
            pltpu.VMEM((16,), f32),
            pltpu.VMEM((128,), f32),
            pltpu.VMEM((128,), f32),
            pltpu.VMEM((128,), f32),
            pltpu.VMEM((8, 128), f32),
            pltpu.VMEM((64,), f32),
            pltpu.VMEM((16,), f32),
            pltpu.VMEM((16,), f32),
            pltpu.VMEM((256,), f32),
            pltpu.VMEM((128,), f32),
            pltpu.VMEM((64,), f32),
            pltpu.VMEM_SHARED((256,), f32),
            pltpu.VMEM_SHARED((128,), f32),
            pltpu.SemaphoreType.DMA,
            pltpu.SemaphoreType.DMA,
        ],
    )(_sc_body)
    return run(z1d, W1v, b1_eff, a, lnw, lnb, W2, b2).reshape(1, 64)


def kernel(z_refined, trust_form, trust_role, W1, b1, prelu_a, ln_w, ln_b, W2, b2):
    b1_eff = b1 + trust_form * W1[:, 4096] + trust_role * W1[:, 4097]
    return _gate(z_refined.reshape(4096), W1.reshape(16, 8, 4098), b1_eff,
                 prelu_a, ln_w, ln_b, W2, b2)


# speedup vs baseline: 1.4848x; 1.4848x over previous
import functools

import jax
import jax.numpy as jnp
from jax import lax
from jax.experimental import pallas as pl
from jax.experimental.pallas import tpu as pltpu
from jax.experimental.pallas import tpu_sc as plsc

_NS = 16
_RPW = 8
_K = 4096
_UNROLL = 4

_mesh = plsc.VectorSubcoreMesh(
    core_axis_name="c", subcore_axis_name="s", num_cores=1)


def _rsqrt_vec(v):
    i = plsc.bitcast(v, jnp.int32)
    i = jnp.full((16,), 0x5F3759DF, jnp.int32) - lax.shift_right_logical(
        i, jnp.full((16,), 1, jnp.int32))
    y = plsc.bitcast(i, jnp.float32)
    half = v * 0.5
    for _ in range(4):
        y = y * (1.5 - half * y * y)
    return y


def _sc_body(z_hbm, w1_hbm, prm_hbm, w2_hbm, out_hbm,
             rows_v, x_v, prm_v, w2_v,
             hseg_v, lseg_v, ep_v, lg_v, out_v, shared_h, shared_l,
             sem_x, sem_rows):
    s = lax.axis_index("s")
    lane = lax.iota(jnp.int32, 16)

    cp_x = pltpu.make_async_copy(z_hbm, x_v, sem_x)
    cp_x.start()
    cp_rows = pltpu.make_async_copy(
        w1_hbm.at[pl.ds(s * _RPW, _RPW), pl.ds(0, _K)], rows_v, sem_rows)
    cp_rows.start()
    pltpu.sync_copy(prm_hbm, prm_v)
    cp_x.wait()
    cp_rows.wait()

    def dot_body(i, accs):
        out = list(accs)
        for u in range(_UNROLL):
            k = i * _UNROLL + u
            xk = x_v[pl.ds(k * 16, 16)]
            for j in range(_RPW):
                out[j] = out[j] + xk * rows_v[j, pl.ds(k * 16, 16)]
        return tuple(out)

    accs = lax.fori_loop(
        0, _K // 16 // _UNROLL, dot_body,
        tuple(jnp.zeros((16,), jnp.float32) for _ in range(_RPW)),
        unroll=False)

    hvec = jnp.zeros((16,), jnp.float32)
    for j in range(_RPW):
        hvec = jnp.where(lane == j, jnp.sum(accs[j]), hvec)

    b1g = plsc.load_gather(prm_v, [s * _RPW + (lane & 7)])
    hvec = hvec + b1g
    a_s = prm_v[pl.ds(448, 16)][0]
    av = jnp.full((16,), a_s, jnp.float32)
    hvec = jnp.maximum(hvec, 0.0) + av * jnp.minimum(hvec, 0.0)

    hseg_v[...] = hvec
    pltpu.sync_copy(hseg_v, shared_h.at[pl.ds(s * 16, 16)])
    plsc.subcore_barrier()

    pltpu.sync_copy(shared_h, ep_v)
    hs = []
    for m in range(8):
        gidx = m * 16 + lane
        hs.append(plsc.load_gather(ep_v, [((gidx >> 3) << 4) + (gidx & 7)]))
    hsum = hs[0]
    sumsq = hs[0] * hs[0]
    for m in range(1, 8):
        hsum = hsum + hs[m]
        sumsq = sumsq + hs[m] * hs[m]
    mu = jnp.sum(hsum) * (1.0 / 128.0)
    var = jnp.sum(sumsq) * (1.0 / 128.0) - mu * mu
    rstd = _rsqrt_vec(jnp.full((16,), var + 1e-5, jnp.float32))
    mu_vec = jnp.full((16,), mu, jnp.float32)
    hn = []
    for m in range(8):
        hn.append((hs[m] - mu_vec) * rstd
                  * prm_v[pl.ds(128 + m * 16, 16)]
                  + prm_v[pl.ds(256 + m * 16, 16)])

    @pl.when(s < 8)
    def _logits():
        pltpu.sync_copy(w2_hbm.at[pl.ds(s * 8, 8), :], w2_v)
        lvec = jnp.zeros((16,), jnp.float32)
        for i in range(8):
            acc = hn[0] * w2_v[i, pl.ds(0, 16)]
            for m in range(1, 8):
                acc = acc + hn[m] * w2_v[i, pl.ds(m * 16, 16)]
            lvec = jnp.where(lane == i, jnp.sum(acc), lvec)
        b2g = plsc.load_gather(prm_v, [384 + s * 8 + (lane & 7)])
        lseg_v[...] = lvec + b2g
        pltpu.sync_copy(lseg_v, shared_l.at[pl.ds(s * 16, 16)])

    plsc.subcore_barrier()

    @pl.when(s == 0)
    def _epilogue():
        pltpu.sync_copy(shared_l, lg_v)
        ls = []
        for t in range(4):
            gidx = t * 16 + lane
            ls.append(plsc.load_gather(lg_v, [((gidx >> 3) << 4) + (gidx & 7)]))
        mx = jnp.maximum(jnp.maximum(ls[0], ls[1]), jnp.maximum(ls[2], ls[3]))
        mxs = jnp.full((16,), jnp.max(mx), jnp.float32)
        es = [jnp.exp(l - mxs) for l in ls]
        tot = jnp.sum(es[0] + es[1] + es[2] + es[3])
        inv = jnp.full((16,), 1.0, jnp.float32) / jnp.full((16,), tot, jnp.float32)
        for t in range(4):
            out_v[pl.ds(t * 16, 16)] = es[t] * inv
        pltpu.sync_copy(out_v, out_hbm)


@jax.jit
def _gate(z, trust_form, trust_role, W1, b1, prelu_a, ln_w, ln_b, W2, b2):
    f32 = jnp.float32
    b1_eff = b1 + trust_form * W1[:, 4096] + trust_role * W1[:, 4097]
    prm = jnp.concatenate(
        [b1_eff, ln_w, ln_b, b2, prelu_a, jnp.zeros((15,), f32)])
    run = functools.partial(
        pl.kernel,
        out_type=jax.ShapeDtypeStruct((64,), f32),
        mesh=_mesh,
        compiler_params=pltpu.CompilerParams(
            needs_layout_passes=False, use_tc_tiling_on_sc=True),
        scratch_types=[
            pltpu.VMEM((_RPW, _K), f32),
            pltpu.VMEM((_K,), f32),
            pltpu.VMEM((464,), f32),
            pltpu.VMEM((8, 128), f32),
            pltpu.VMEM((16,), f32),
            pltpu.VMEM((16,), f32),
            pltpu.VMEM((256,), f32),
            pltpu.VMEM((128,), f32),
            pltpu.VMEM((64,), f32),
            pltpu.VMEM_SHARED((256,), f32),
            pltpu.VMEM_SHARED((128,), f32),
            pltpu.SemaphoreType.DMA,
            pltpu.SemaphoreType.DMA,
        ],
    )(_sc_body)
    return run(z.reshape(4096), W1, prm, W2).reshape(1, 64)


def kernel(z_refined, trust_form, trust_role, W1, b1, prelu_a, ln_w, ln_b, W2, b2):
    return _gate(z_refined, trust_form, trust_role, W1, b1, prelu_a,
                 ln_w, ln_b, W2, b2)


# speedup vs baseline: 4.7144x; 3.1751x over previous
import jax
import jax.numpy as jnp
from jax.experimental import pallas as pl
from jax.experimental.pallas import tpu as pltpu


def _dotT(a, b):
    return jax.lax.dot_general(
        a, b, (((1,), (1,)), ((), ())), preferred_element_type=jnp.float32
    )


def _gate_body(z_hbm, tf_hbm, tr_hbm, w1_hbm, b1_hbm, a_hbm,
               lnw_hbm, lnb_hbm, w2_hbm, b2_hbm, out_ref,
               z_v, h0_v, h1_v, tf_v, tr_v, b1_v, a_v, lnw_v, lnb_v,
               w2_v, b2_v, sem_z, sem_h0, sem_h1, sem_small):
    cp_z = pltpu.make_async_copy(z_hbm, z_v, sem_z)
    cp_z.start()
    cp_h0 = pltpu.make_async_copy(w1_hbm.at[pl.ds(0, 64), :], h0_v, sem_h0)
    cp_h0.start()
    cp_h1 = pltpu.make_async_copy(w1_hbm.at[pl.ds(64, 64), :], h1_v, sem_h1)
    cp_h1.start()
    small = [
        pltpu.make_async_copy(tf_hbm, tf_v, sem_small),
        pltpu.make_async_copy(tr_hbm, tr_v, sem_small),
        pltpu.make_async_copy(b1_hbm, b1_v, sem_small),
        pltpu.make_async_copy(a_hbm, a_v, sem_small),
        pltpu.make_async_copy(lnw_hbm, lnw_v, sem_small),
        pltpu.make_async_copy(lnb_hbm, lnb_v, sem_small),
        pltpu.make_async_copy(w2_hbm, w2_v, sem_small),
        pltpu.make_async_copy(b2_hbm, b2_v, sem_small),
    ]
    for cp in small:
        cp.start()

    cp_z.wait()
    cp_h0.wait()
    seg0 = _dotT(z_v[...], h0_v[:, 0:4096])
    cp_h1.wait()
    seg1 = _dotT(z_v[...], h1_v[:, 0:4096])
    h = jnp.concatenate([seg0, seg1], axis=1)

    for cp in small:
        cp.wait()
    t0 = jnp.concatenate([h0_v[:, 4096:4097], h1_v[:, 4096:4097]], axis=0)
    t1 = jnp.concatenate([h0_v[:, 4097:4098], h1_v[:, 4097:4098]], axis=0)
    h = h + _dotT(tf_v[...], t0)
    h = h + _dotT(tr_v[...], t1)
    h = h + b1_v[...]
    h = jnp.maximum(h, 0.0) + a_v[...] * jnp.minimum(h, 0.0)
    mu = jnp.mean(h, axis=1, keepdims=True)
    d = h - mu
    var = jnp.mean(d * d, axis=1, keepdims=True)
    hn = d * jax.lax.rsqrt(var + 1e-5) * lnw_v[...] + lnb_v[...]
    logits = _dotT(hn, w2_v[...]) + b2_v[...]
    m = jnp.max(logits, axis=1, keepdims=True)
    e = jnp.exp(logits - m)
    s = jnp.sum(e, axis=1, keepdims=True)
    out_ref[...] = e / s


@jax.jit
def _gate(z, tf, tr, W1, b1, a, lnw, lnb, W2, b2):
    hbm = pl.BlockSpec(memory_space=pltpu.MemorySpace.HBM)
    return pl.pallas_call(
        _gate_body,
        out_shape=jax.ShapeDtypeStruct((1, 64), jnp.float32),
        in_specs=[hbm] * 10,
        out_specs=pl.BlockSpec(memory_space=pltpu.MemorySpace.VMEM),
        scratch_shapes=[
            pltpu.VMEM((1, 4096), jnp.float32),
            pltpu.VMEM((64, 4098), jnp.float32),
            pltpu.VMEM((64, 4098), jnp.float32),
            pltpu.VMEM((1, 1), jnp.float32),
            pltpu.VMEM((1, 1), jnp.float32),
            pltpu.VMEM((1, 128), jnp.float32),
            pltpu.VMEM((1, 1), jnp.float32),
            pltpu.VMEM((1, 128), jnp.float32),
            pltpu.VMEM((1, 128), jnp.float32),
            pltpu.VMEM((64, 128), jnp.float32),
            pltpu.VMEM((1, 64), jnp.float32),
            pltpu.SemaphoreType.DMA,
            pltpu.SemaphoreType.DMA,
            pltpu.SemaphoreType.DMA,
            pltpu.SemaphoreType.DMA,
        ],
    )(z, tf, tr, W1, b1, a, lnw, lnb, W2, b2)


def kernel(z_refined, trust_form, trust_role, W1, b1, prelu_a, ln_w, ln_b, W2, b2):
    return _gate(
        z_refined,
        trust_form.reshape(1, 1),
        trust_role.reshape(1, 1),
        W1,
        b1.reshape(1, 128),
        prelu_a.reshape(1, 1),
        ln_w.reshape(1, 128),
        ln_b.reshape(1, 128),
        W2,
        b2.reshape(1, 64),
    )


# speedup vs baseline: 4.8827x; 1.0357x over previous
import jax
import jax.numpy as jnp
from jax.experimental import pallas as pl
from jax.experimental.pallas import tpu as pltpu


def _dotT(a, b):
    return jax.lax.dot_general(
        a, b, (((1,), (1,)), ((), ())), preferred_element_type=jnp.float32
    )


def _gate_body(tf_s, tr_s, a_s, z_hbm, w1_hbm, b1_hbm,
               lnw_hbm, lnb_hbm, w2_hbm, b2_hbm, out_ref,
               w1_v, z_v, b1_v, lnw_v, lnb_v, w2_v, b2_v,
               sem_w1, sem_small):
    cp_w1 = pltpu.make_async_copy(w1_hbm, w1_v, sem_w1)
    cp_w1.start()
    small = [
        pltpu.make_async_copy(z_hbm, z_v, sem_small),
        pltpu.make_async_copy(b1_hbm, b1_v, sem_small),
        pltpu.make_async_copy(lnw_hbm, lnw_v, sem_small),
        pltpu.make_async_copy(lnb_hbm, lnb_v, sem_small),
        pltpu.make_async_copy(w2_hbm, w2_v, sem_small),
        pltpu.make_async_copy(b2_hbm, b2_v, sem_small),
    ]
    for cp in small:
        cp.start()
    for cp in small:
        cp.wait()
    cp_w1.wait()

    h = _dotT(z_v[...], w1_v[:, 0:4096])
    h = h + _dotT(jnp.full((1, 1), tf_s[0], jnp.float32), w1_v[:, 4096:4097])
    h = h + _dotT(jnp.full((1, 1), tr_s[0], jnp.float32), w1_v[:, 4097:4098])
    h = h + b1_v[...]
    h = jnp.maximum(h, 0.0) + a_s[0] * jnp.minimum(h, 0.0)
    mu = jnp.mean(h, axis=1, keepdims=True)
    d = h - mu
    var = jnp.mean(d * d, axis=1, keepdims=True)
    hn = d * jax.lax.rsqrt(var + 1e-5) * lnw_v[...] + lnb_v[...]
    logits = _dotT(hn, w2_v[...]) + b2_v[...]
    m = jnp.max(logits, axis=1, keepdims=True)
    e = jnp.exp(logits - m)
    s = jnp.sum(e, axis=1, keepdims=True)
    out_ref[...] = e / s


@jax.jit
def _gate(tf, tr, a, z, W1, b1, lnw, lnb, W2, b2):
    hbm = pl.BlockSpec(memory_space=pltpu.MemorySpace.HBM)
    smem = pl.BlockSpec(memory_space=pltpu.MemorySpace.SMEM)
    return pl.pallas_call(
        _gate_body,
        out_shape=jax.ShapeDtypeStruct((1, 64), jnp.float32),
        in_specs=[smem, smem, smem] + [hbm] * 7,
        out_specs=pl.BlockSpec(memory_space=pltpu.MemorySpace.VMEM),
        scratch_shapes=[
            pltpu.VMEM((128, 4098), jnp.float32),
            pltpu.VMEM((1, 4096), jnp.float32),
            pltpu.VMEM((1, 128), jnp.float32),
            pltpu.VMEM((1, 128), jnp.float32),
            pltpu.VMEM((1, 128), jnp.float32),
            pltpu.VMEM((64, 128), jnp.float32),
            pltpu.VMEM((1, 64), jnp.float32),
            pltpu.SemaphoreType.DMA,
            pltpu.SemaphoreType.DMA,
        ],
    )(tf, tr, a, z, W1, b1, lnw, lnb, W2, b2)


def kernel(z_refined, trust_form, trust_role, W1, b1, prelu_a, ln_w, ln_b, W2, b2):
    return _gate(
        trust_form,
        trust_role,
        prelu_a,
        z_refined,
        W1,
        b1.reshape(1, 128),
        ln_w.reshape(1, 128),
        ln_b.reshape(1, 128),
        W2,
        b2.reshape(1, 64),
    )


# speedup vs baseline: 4.9534x; 1.0145x over previous
import jax
import jax.numpy as jnp
from jax.experimental import pallas as pl
from jax.experimental.pallas import tpu as pltpu


def _dotT(a, b):
    return jax.lax.dot_general(
        a, b, (((1,), (1,)), ((), ())), preferred_element_type=jnp.float32
    )


def _gate_body(z_hbm, tf_hbm, tr_hbm, w1_hbm, b1_hbm, a_hbm,
               lnw_hbm, lnb_hbm, w2_hbm, b2_hbm, out_ref,
               w1_v, z_v, tf_v, tr_v, b1_v, a_v, lnw_v, lnb_v, w2_v, b2_v,
               sem_w1, sem_small):
    cp_w1 = pltpu.make_async_copy(w1_hbm, w1_v, sem_w1)
    cp_w1.start()
    small = [
        pltpu.make_async_copy(z_hbm, z_v, sem_small),
        pltpu.make_async_copy(tf_hbm, tf_v, sem_small),
        pltpu.make_async_copy(tr_hbm, tr_v, sem_small),
        pltpu.make_async_copy(b1_hbm, b1_v, sem_small),
        pltpu.make_async_copy(a_hbm, a_v, sem_small),
        pltpu.make_async_copy(lnw_hbm, lnw_v, sem_small),
        pltpu.make_async_copy(lnb_hbm, lnb_v, sem_small),
        pltpu.make_async_copy(w2_hbm, w2_v, sem_small),
        pltpu.make_async_copy(b2_hbm, b2_v, sem_small),
    ]
    for cp in small:
        cp.start()
    for cp in small:
        cp.wait()
    cp_w1.wait()

    h = _dotT(z_v[...], w1_v[:, 0:4096])
    h = h + _dotT(tf_v[...], w1_v[:, 4096:4097])
    h = h + _dotT(tr_v[...], w1_v[:, 4097:4098])
    h = h + b1_v[...]
    h = jnp.maximum(h, 0.0) + a_v[...] * jnp.minimum(h, 0.0)
    mu = jnp.mean(h, axis=1, keepdims=True)
    d = h - mu
    var = jnp.mean(d * d, axis=1, keepdims=True)
    hn = d * jax.lax.rsqrt(var + 1e-5) * lnw_v[...] + lnb_v[...]
    logits = _dotT(hn, w2_v[...]) + b2_v[...]
    m = jnp.max(logits, axis=1, keepdims=True)
    e = jnp.exp(logits - m)
    s = jnp.sum(e, axis=1, keepdims=True)
    out_ref[...] = e / s


@jax.jit
def _gate(z, tf, tr, W1, b1, a, lnw, lnb, W2, b2):
    hbm = pl.BlockSpec(memory_space=pltpu.MemorySpace.HBM)
    return pl.pallas_call(
        _gate_body,
        out_shape=jax.ShapeDtypeStruct((1, 64), jnp.float32),
        in_specs=[hbm] * 10,
        out_specs=pl.BlockSpec(memory_space=pltpu.MemorySpace.VMEM),
        scratch_shapes=[
            pltpu.VMEM((128, 4098), jnp.float32),
            pltpu.VMEM((1, 4096), jnp.float32),
            pltpu.VMEM((1, 1), jnp.float32),
            pltpu.VMEM((1, 1), jnp.float32),
            pltpu.VMEM((1, 128), jnp.float32),
            pltpu.VMEM((1, 1), jnp.float32),
            pltpu.VMEM((1, 128), jnp.float32),
            pltpu.VMEM((1, 128), jnp.float32),
            pltpu.VMEM((64, 128), jnp.float32),
            pltpu.VMEM((1, 64), jnp.float32),
            pltpu.SemaphoreType.DMA,
            pltpu.SemaphoreType.DMA,
        ],
    )(z, tf, tr, W1, b1, a, lnw, lnb, W2, b2)


def kernel(z_refined, trust_form, trust_role, W1, b1, prelu_a, ln_w, ln_b, W2, b2):
    return _gate(
        z_refined,
        trust_form.reshape(1, 1),
        trust_role.reshape(1, 1),
        W1,
        b1.reshape(1, 128),
        prelu_a.reshape(1, 1),
        ln_w.reshape(1, 128),
        ln_b.reshape(1, 128),
        W2,
        b2.reshape(1, 64),
    )
